# baseline (speedup 1.0000x reference)
import functools

import jax
import jax.numpy as jnp
from jax import lax
from jax.experimental import pallas as pl
from jax.experimental.pallas import tpu as pltpu
from jax.experimental.pallas import tpu_sc as plsc

N = 10000
E = 320000
HID = 64
NH = 2
DH = HID // NH


BQ = 400


def _attn_body(q_ref, kt_ref, v_ref, o_ref, *, scale):
    q = q_ref[0]
    kt = kt_ref[0]
    v = v_ref[0]
    s = jnp.dot(q, kt, preferred_element_type=jnp.float32) * scale
    m = jnp.max(s, axis=1, keepdims=True)
    p = jnp.exp(s - m)
    l = jnp.sum(p, axis=1, keepdims=True)
    o = jnp.dot(p, v, preferred_element_type=jnp.float32) / l
    o_ref[0] = o


def _attention(q, k, v):
    kt = jnp.swapaxes(k, 1, 2)
    return pl.pallas_call(
        functools.partial(_attn_body, scale=float(1.0 / (DH ** 0.5))),
        grid=(NH, N // BQ),
        in_specs=[
            pl.BlockSpec((1, BQ, DH), lambda h, i: (h, i, 0)),
            pl.BlockSpec((1, DH, N), lambda h, i: (h, 0, 0)),
            pl.BlockSpec((1, N, DH), lambda h, i: (h, 0, 0)),
        ],
        out_specs=pl.BlockSpec((1, BQ, DH), lambda h, i: (h, i, 0)),
        out_shape=jax.ShapeDtypeStruct((NH, N, DH), jnp.float32),
    )(q, kt, v)


CHUNK = 64
NCHUNK = E // CHUNK
NWORK = 32
NT = NCHUNK // NWORK
NTMAX = NT + 1
NEXTRA = NCHUNK - NWORK * NT
NHALF = N // 2
RBASE = 312
RLEN = 64
RCOPIES = 5


def _edge_body(write_ehat, av_hbm, bh_hbm, ce_hbm, src_hbm, dst_hbm,
               nd_hbm, ehat_hbm,
               sia_v, dia_v, di2_v, av0_v, av1_v, bh0_v, bh1_v,
               ce0_v, ce1_v, ms_v, nd_sp, sem0, sem1):
    c = lax.axis_index("c")
    s = lax.axis_index("s")
    w = s * 2 + c
    c0 = w * NT + jnp.minimum(w, NEXTRA)
    nch = jnp.where(w < NEXTRA, NTMAX, NT)
    base0 = c0 * CHUNK
    zero = jnp.zeros((16,), jnp.float32)

    pltpu.sync_copy(src_hbm.at[pl.ds(base0, NT * CHUNK)],
                    sia_v.at[pl.ds(0, NT * CHUNK)])
    pltpu.sync_copy(dst_hbm.at[pl.ds(base0, NT * CHUNK)],
                    dia_v.at[pl.ds(0, NT * CHUNK)])

    @pl.when(w < NEXTRA)
    def _():
        pltpu.sync_copy(src_hbm.at[pl.ds(base0 + NT * CHUNK, CHUNK)],
                        sia_v.at[pl.ds(NT * CHUNK, CHUNK)])
        pltpu.sync_copy(dst_hbm.at[pl.ds(base0 + NT * CHUNK, CHUNK)],
                        dia_v.at[pl.ds(NT * CHUNK, CHUNK)])

    sets = ((av0_v, bh0_v, ce0_v, sem0), (av1_v, bh1_v, ce1_v, sem1))

    def _issue(t, bset):
        av_v, bh_v, ce_v, sem = bset
        pltpu.async_copy(av_hbm.at[sia_v.at[pl.ds(t * CHUNK, CHUNK)]],
                         av_v, sem)
        pltpu.async_copy(bh_hbm.at[dia_v.at[pl.ds(t * CHUNK, CHUNK)]],
                         bh_v, sem)
        pltpu.async_copy(ce_hbm.at[pl.ds((c0 + t) * CHUNK, CHUNK)], ce_v, sem)

    def _wait(bset):
        av_v, bh_v, ce_v, sem = bset
        pltpu.make_async_copy(av_hbm.at[sia_v.at[pl.ds(0, CHUNK)]],
                              av_v, sem).wait()
        pltpu.make_async_copy(bh_hbm.at[dia_v.at[pl.ds(0, CHUNK)]],
                              bh_v, sem).wait()
        pltpu.make_async_copy(ce_hbm.at[pl.ds(0, CHUNK)], ce_v, sem).wait()

    def _process(p, h0, t, bset):
        av_v, bh_v, ce_v, _ = bset

        def _idx(i, _):
            dv = dia_v[pl.ds(t * CHUNK + 16 * i, 16)]
            rel = dv - h0
            ok = (rel >= 0) & (rel < NHALF)
            di2_v[pl.ds(16 * i, 16)] = jnp.where(ok, rel, NHALF)
            return 0

        lax.fori_loop(0, CHUNK // 16, _idx, 0)

        def _row(r, _):
            for j in range(4):
                a = av_v[r, pl.ds(16 * j, 16)]
                vv = av_v[r, pl.ds(64 + 16 * j, 16)]
                b = bh_v[r, pl.ds(16 * j, 16)]
                cc = ce_v[r, pl.ds(16 * j, 16)]
                ehat = a + b + cc
                if write_ehat and p == 0:
                    ce_v[r, pl.ds(16 * j, 16)] = ehat
                sg = 1.0 / (1.0 + jnp.exp(-ehat))
                ms_v[r, pl.ds(64 + 16 * j, 16)] = sg
                ms_v[r, pl.ds(16 * j, 16)] = sg * vv
            return 0

        lax.fori_loop(0, CHUNK, _row, 0, unroll=2)

        if write_ehat and p == 0:
            pltpu.sync_copy(ce_v, ehat_hbm.at[pl.ds((c0 + t) * CHUNK, CHUNK)])
        pltpu.sync_copy(ms_v, nd_sp.at[di2_v], add=True)

    for p in range(2):
        h0 = p * NHALF
        plsc.subcore_barrier()

        def _zero_buf(r, _):
            for j in range(8):
                ms_v[r, pl.ds(16 * j, 16)] = zero
            return 0

        lax.fori_loop(0, RLEN, _zero_buf, 0)
        for k in range(RCOPIES):
            pltpu.sync_copy(ms_v.at[pl.ds(0, RLEN)],
                            nd_sp.at[pl.ds(s * RBASE + k * RLEN, RLEN)])
        plsc.subcore_barrier()

        _issue(0, sets[0])

        def _pair(u, _):
            tA = 2 * u
            tB = 2 * u + 1

            @pl.when(tB < nch)
            def _():
                _issue(tB, sets[1])

            @pl.when(tA < nch)
            def _():
                _wait(sets[0])
                _process(p, h0, tA, sets[0])

            @pl.when(tB + 1 < nch)
            def _():
                _issue(tB + 1, sets[0])

            @pl.when(tB < nch)
            def _():
                _wait(sets[1])
                _process(p, h0, tB, sets[1])

            return 0

        lax.fori_loop(0, (NTMAX + 1) // 2, _pair, 0)
        plsc.subcore_barrier()

        for k in range(RCOPIES):
            r0 = s * RBASE + k * RLEN
            pltpu.sync_copy(nd_sp.at[pl.ds(r0, RLEN)], ms_v.at[pl.ds(0, RLEN)])
            pltpu.sync_copy(ms_v.at[pl.ds(0, RLEN)],
                            nd_hbm.at[c, pl.ds(h0 + r0, RLEN)])


def _edge_call(write_ehat):
    out_type = [jax.ShapeDtypeStruct((2, N, 2 * HID), jnp.float32)]
    if write_ehat:
        out_type.append(jax.ShapeDtypeStruct((E, HID), jnp.float32))
    body = functools.partial(_edge_body, write_ehat)
    if not write_ehat:
        def body(av, bh, ce, src, dst, nd, *rest):
            return _edge_body(False, av, bh, ce, src, dst, nd, None, *rest)
    return pl.kernel(
        body,
        mesh=plsc.VectorSubcoreMesh(core_axis_name="c", subcore_axis_name="s"),
        out_type=out_type,
        scratch_types=[
            pltpu.VMEM((NTMAX * CHUNK,), jnp.int32),
            pltpu.VMEM((NTMAX * CHUNK,), jnp.int32),
            pltpu.VMEM((CHUNK,), jnp.int32),
            pltpu.VMEM((CHUNK, 2 * HID), jnp.float32),
            pltpu.VMEM((CHUNK, 2 * HID), jnp.float32),
            pltpu.VMEM((CHUNK, 2 * HID), jnp.float32),
            pltpu.VMEM((CHUNK, 2 * HID), jnp.float32),
            pltpu.VMEM((CHUNK, HID), jnp.float32),
            pltpu.VMEM((CHUNK, HID), jnp.float32),
            pltpu.VMEM((CHUNK, 2 * HID), jnp.float32),
            pltpu.VMEM_SHARED((NHALF + 8, 2 * HID), jnp.float32),
            pltpu.SemaphoreType.DMA,
            pltpu.SemaphoreType.DMA,
        ],
    )


def _edge_stage(Ah, Bh, Vh, Ce, src, dst, write_ehat):
    av = jnp.concatenate([Ah, Vh], axis=1)
    bb = jnp.concatenate([Bh, Bh], axis=1)
    outs = _edge_call(write_ehat)(av, bb, Ce, src, dst)
    nd = outs[0][0] + outs[0][1]
    num = nd[:, :HID]
    den = nd[:, HID:] + 1e-6
    return num, den, (outs[1] if write_ehat else None)


def _ln(x):
    m = jnp.mean(x, axis=-1, keepdims=True)
    v = jnp.var(x, axis=-1, keepdims=True)
    return (x - m) / jnp.sqrt(v + 1e-5)


def kernel(x, pos_enc, edge_attr, edge_index, Wn, Wp, b0, We, be, A, B, Cc, U, V, Wq, Wk, Wv, Wo, W1, b1, W2, b2, P1, pb1, P2, pb2, P3, pb3):
    L = A.shape[0]
    src = edge_index[0]
    dst = edge_index[1]
    h = x @ Wn + pos_enc @ Wp + b0
    e = edge_attr @ We + be
    for l in range(L):
        Ah = h @ A[l]
        Bh = h @ B[l]
        Vh = h @ V[l]
        Ce = e @ Cc[l]
        last = (l + 1 == L)
        num, den, e_hat = _edge_stage(Ah, Bh, Vh, Ce, src, dst,
                                      write_ehat=not last)
        h_local = h @ U[l] + num / den
        if not last:
            e = _ln(e + e_hat)
        q = jnp.transpose((h @ Wq[l]).reshape(N, NH, DH), (1, 0, 2))
        k = jnp.transpose((h @ Wk[l]).reshape(N, NH, DH), (1, 0, 2))
        v = jnp.transpose((h @ Wv[l]).reshape(N, NH, DH), (1, 0, 2))
        o = _attention(q, k, v)
        h_attn = jnp.transpose(o, (1, 0, 2)).reshape(N, HID) @ Wo[l]
        h = _ln(h + h_local) + _ln(h + h_attn)
        h = _ln(h + jax.nn.relu(h @ W1[l] + b1[l]) @ W2[l] + b2[l])
    hg = jnp.sum(h, axis=0, keepdims=True)
    z = jax.nn.relu(hg @ P1 + pb1)
    z = jax.nn.relu(z @ P2 + pb2)
    return z @ P3 + pb3


# speedup vs baseline: 1.6994x; 1.6994x over previous
import functools

import jax
import jax.numpy as jnp
from jax import lax
from jax.experimental import pallas as pl
from jax.experimental.pallas import tpu as pltpu
from jax.experimental.pallas import tpu_sc as plsc

N = 10000
E = 320000
HID = 64
NH = 2
DH = HID // NH


BQ = 400


def _attn_body(q_ref, kt_ref, v_ref, o_ref, *, scale):
    q = q_ref[0]
    kt = kt_ref[0]
    v = v_ref[0]
    s = jnp.dot(q, kt, preferred_element_type=jnp.float32) * scale
    m = jnp.max(s, axis=1, keepdims=True)
    p = jnp.exp(s - m)
    l = jnp.sum(p, axis=1, keepdims=True)
    o = jnp.dot(p, v, preferred_element_type=jnp.float32) / l
    o_ref[0] = o


def _attention(q, k, v):
    kt = jnp.swapaxes(k, 1, 2)
    return pl.pallas_call(
        functools.partial(_attn_body, scale=float(1.0 / (DH ** 0.5))),
        grid=(NH, N // BQ),
        in_specs=[
            pl.BlockSpec((1, BQ, DH), lambda h, i: (h, i, 0)),
            pl.BlockSpec((1, DH, N), lambda h, i: (h, 0, 0)),
            pl.BlockSpec((1, N, DH), lambda h, i: (h, 0, 0)),
        ],
        out_specs=pl.BlockSpec((1, BQ, DH), lambda h, i: (h, i, 0)),
        out_shape=jax.ShapeDtypeStruct((NH, N, DH), jnp.float32),
    )(q, kt, v)


CHUNK = 128
NCHUNK = E // CHUNK
NWORK = 32
NT = NCHUNK // NWORK
NTMAX = NT + 1
NEXTRA = NCHUNK - NWORK * NT
NHALF = N // 2
RBASE = 312
RLEN = 64
RCOPIES = 5


def _edge_body(write_ehat, av_hbm, bh_hbm, ce_hbm, src_hbm, dst_hbm,
               nd_hbm, ehat_hbm,
               si0_v, di0_v, si1_v, di1_v, di2_v,
               av0_v, av1_v, bh0_v, bh1_v, ce_v, nd_sp,
               semg0, semg1, semi0, semi1):
    c = lax.axis_index("c")
    s = lax.axis_index("s")
    w = s * 2 + c
    c0 = w * NT + jnp.minimum(w, NEXTRA)
    nch = jnp.where(w < NEXTRA, NTMAX, NT)
    zero = jnp.zeros((16,), jnp.float32)

    isets = ((si0_v, di0_v, semi0), (si1_v, di1_v, semi1))
    gsets = ((av0_v, bh0_v, semg0), (av1_v, bh1_v, semg1))

    def _issue_idx(t, k):
        si_v, di_v, sem = isets[k]
        base = (c0 + t) * CHUNK
        pltpu.async_copy(src_hbm.at[pl.ds(base, CHUNK)], si_v, sem)
        pltpu.async_copy(dst_hbm.at[pl.ds(base, CHUNK)], di_v, sem)

    def _wait_idx(k):
        si_v, di_v, sem = isets[k]
        pltpu.make_async_copy(src_hbm.at[pl.ds(0, CHUNK)], si_v, sem).wait()
        pltpu.make_async_copy(dst_hbm.at[pl.ds(0, CHUNK)], di_v, sem).wait()

    def _issue_gather(k):
        si_v, di_v, _ = isets[k]
        av_v, bh_v, sem = gsets[k]
        pltpu.async_copy(av_hbm.at[si_v], av_v, sem)
        pltpu.async_copy(bh_hbm.at[di_v], bh_v, sem)

    def _wait_gather(k):
        si_v, di_v, _ = isets[k]
        av_v, bh_v, sem = gsets[k]
        pltpu.make_async_copy(av_hbm.at[si_v], av_v, sem).wait()
        pltpu.make_async_copy(bh_hbm.at[di_v], bh_v, sem).wait()

    def _process(p, h0, t, k):
        di_v = isets[k][1]
        av_v, bh_v, _ = gsets[k]
        base = (c0 + t) * CHUNK

        pltpu.sync_copy(ce_hbm.at[pl.ds(base, CHUNK)], ce_v)

        def _idx(i, _):
            dv = di_v[pl.ds(16 * i, 16)]
            rel = dv - h0
            ok = (rel >= 0) & (rel < NHALF)
            di2_v[pl.ds(16 * i, 16)] = jnp.where(ok, rel, NHALF)
            return 0

        lax.fori_loop(0, CHUNK // 16, _idx, 0)

        def _row(r, _):
            for j in range(4):
                a = av_v[r, pl.ds(16 * j, 16)]
                vv = av_v[r, pl.ds(64 + 16 * j, 16)]
                b = bh_v[r, pl.ds(16 * j, 16)]
                cc = ce_v[r, pl.ds(16 * j, 16)]
                ehat = a + b + cc
                if write_ehat and p == 0:
                    ce_v[r, pl.ds(16 * j, 16)] = ehat
                sg = 1.0 / (1.0 + jnp.exp(-ehat))
                av_v[r, pl.ds(64 + 16 * j, 16)] = sg
                av_v[r, pl.ds(16 * j, 16)] = sg * vv
            return 0

        lax.fori_loop(0, CHUNK, _row, 0)

        if write_ehat and p == 0:
            pltpu.sync_copy(ce_v, ehat_hbm.at[pl.ds(base, CHUNK)])
        pltpu.sync_copy(av_v, nd_sp.at[di2_v], add=True)

    def _half(p, h0, t, k):
        kn = 1 - k

        @pl.when(t + 1 < nch)
        def _():
            _wait_idx(kn)
            _issue_gather(kn)

        @pl.when(t < nch)
        def _():
            _wait_gather(k)

        @pl.when(t + 2 < nch)
        def _():
            _issue_idx(t + 2, k)

        @pl.when(t < nch)
        def _():
            _process(p, h0, t, k)

    for p in range(2):
        h0 = p * NHALF
        plsc.subcore_barrier()

        def _zero_buf(r, _):
            for j in range(8):
                av0_v[r, pl.ds(16 * j, 16)] = zero
            return 0

        lax.fori_loop(0, RLEN, _zero_buf, 0)
        for k in range(RCOPIES):
            pltpu.sync_copy(av0_v.at[pl.ds(0, RLEN)],
                            nd_sp.at[pl.ds(s * RBASE + k * RLEN, RLEN)])
        plsc.subcore_barrier()

        _issue_idx(0, 0)
        _wait_idx(0)
        _issue_gather(0)
        _issue_idx(1, 1)

        def _pair(u, _):
            _half(p, h0, 2 * u, 0)
            _half(p, h0, 2 * u + 1, 1)
            return 0

        lax.fori_loop(0, (NTMAX + 1) // 2, _pair, 0)
        plsc.subcore_barrier()

        for k in range(RCOPIES):
            r0 = s * RBASE + k * RLEN
            pltpu.sync_copy(nd_sp.at[pl.ds(r0, RLEN)], av0_v.at[pl.ds(0, RLEN)])
            pltpu.sync_copy(av0_v.at[pl.ds(0, RLEN)],
                            nd_hbm.at[c, pl.ds(h0 + r0, RLEN)])


def _edge_call(write_ehat):
    out_type = [jax.ShapeDtypeStruct((2, N, 2 * HID), jnp.float32)]
    if write_ehat:
        out_type.append(jax.ShapeDtypeStruct((E, HID), jnp.float32))
    body = functools.partial(_edge_body, write_ehat)
    if not write_ehat:
        def body(av, bh, ce, src, dst, nd, *rest):
            return _edge_body(False, av, bh, ce, src, dst, nd, None, *rest)
    return pl.kernel(
        body,
        mesh=plsc.VectorSubcoreMesh(core_axis_name="c", subcore_axis_name="s"),
        out_type=out_type,
        scratch_types=[
            pltpu.VMEM((CHUNK,), jnp.int32),
            pltpu.VMEM((CHUNK,), jnp.int32),
            pltpu.VMEM((CHUNK,), jnp.int32),
            pltpu.VMEM((CHUNK,), jnp.int32),
            pltpu.VMEM((CHUNK,), jnp.int32),
            pltpu.VMEM((CHUNK, 2 * HID), jnp.float32),
            pltpu.VMEM((CHUNK, 2 * HID), jnp.float32),
            pltpu.VMEM((CHUNK, 2 * HID), jnp.float32),
            pltpu.VMEM((CHUNK, 2 * HID), jnp.float32),
            pltpu.VMEM((CHUNK, HID), jnp.float32),
            pltpu.VMEM_SHARED((NHALF + 8, 2 * HID), jnp.float32),
            pltpu.SemaphoreType.DMA,
            pltpu.SemaphoreType.DMA,
            pltpu.SemaphoreType.DMA,
            pltpu.SemaphoreType.DMA,
        ],
    )


def _edge_stage(Ah, Bh, Vh, Ce, src, dst, write_ehat):
    av = jnp.concatenate([Ah, Vh], axis=1)
    bb = jnp.concatenate([Bh, Bh], axis=1)
    outs = _edge_call(write_ehat)(av, bb, Ce, src, dst)
    nd = outs[0][0] + outs[0][1]
    num = nd[:, :HID]
    den = nd[:, HID:] + 1e-6
    return num, den, (outs[1] if write_ehat else None)


def _ln(x):
    m = jnp.mean(x, axis=-1, keepdims=True)
    v = jnp.var(x, axis=-1, keepdims=True)
    return (x - m) / jnp.sqrt(v + 1e-5)


def kernel(x, pos_enc, edge_attr, edge_index, Wn, Wp, b0, We, be, A, B, Cc, U, V, Wq, Wk, Wv, Wo, W1, b1, W2, b2, P1, pb1, P2, pb2, P3, pb3):
    L = A.shape[0]
    src = edge_index[0]
    dst = edge_index[1]
    h = x @ Wn + pos_enc @ Wp + b0
    e = edge_attr @ We + be
    for l in range(L):
        Ah = h @ A[l]
        Bh = h @ B[l]
        Vh = h @ V[l]
        Ce = e @ Cc[l]
        last = (l + 1 == L)
        num, den, e_hat = _edge_stage(Ah, Bh, Vh, Ce, src, dst,
                                      write_ehat=not last)
        h_local = h @ U[l] + num / den
        if not last:
            e = _ln(e + e_hat)
        q = jnp.transpose((h @ Wq[l]).reshape(N, NH, DH), (1, 0, 2))
        k = jnp.transpose((h @ Wk[l]).reshape(N, NH, DH), (1, 0, 2))
        v = jnp.transpose((h @ Wv[l]).reshape(N, NH, DH), (1, 0, 2))
        o = _attention(q, k, v)
        h_attn = jnp.transpose(o, (1, 0, 2)).reshape(N, HID) @ Wo[l]
        h = _ln(h + h_local) + _ln(h + h_attn)
        h = _ln(h + jax.nn.relu(h @ W1[l] + b1[l]) @ W2[l] + b2[l])
    hg = jnp.sum(h, axis=0, keepdims=True)
    z = jax.nn.relu(hg @ P1 + pb1)
    z = jax.nn.relu(z @ P2 + pb2)
    return z @ P3 + pb3


# speedup vs baseline: 1.7579x; 1.0344x over previous
import functools

import jax
import jax.numpy as jnp
from jax import lax
from jax.experimental import pallas as pl
from jax.experimental.pallas import tpu as pltpu
from jax.experimental.pallas import tpu_sc as plsc

N = 10000
E = 320000
HID = 64
NH = 2
DH = HID // NH


BQ = 400


def _attn_body(q_ref, kt_ref, v_ref, o_ref, *, scale):
    q = q_ref[0].astype(jnp.bfloat16)
    kt = kt_ref[0].astype(jnp.bfloat16)
    v = v_ref[0].astype(jnp.bfloat16)
    s = jnp.dot(q, kt, preferred_element_type=jnp.float32) * scale
    m = jnp.max(s, axis=1, keepdims=True)
    p = jnp.exp(s - m)
    l = jnp.sum(p, axis=1, keepdims=True)
    o = jnp.dot(p.astype(jnp.bfloat16), v, preferred_element_type=jnp.float32) / l
    o_ref[0] = o


def _attention(q, k, v):
    kt = jnp.swapaxes(k, 1, 2)
    return pl.pallas_call(
        functools.partial(_attn_body, scale=float(1.0 / (DH ** 0.5))),
        grid=(NH, N // BQ),
        in_specs=[
            pl.BlockSpec((1, BQ, DH), lambda h, i: (h, i, 0)),
            pl.BlockSpec((1, DH, N), lambda h, i: (h, 0, 0)),
            pl.BlockSpec((1, N, DH), lambda h, i: (h, 0, 0)),
        ],
        out_specs=pl.BlockSpec((1, BQ, DH), lambda h, i: (h, i, 0)),
        out_shape=jax.ShapeDtypeStruct((NH, N, DH), jnp.float32),
    )(q, kt, v)


CHUNK = 128
NCHUNK = E // CHUNK
NWORK = 32
TMAX = -(-NCHUNK // NWORK)
NHALF = N // 2
RBASE = 312
RLEN = 64
RCOPIES = 5


def _edge_body(write_ehat, av_hbm, bh_hbm, ce_hbm, src_hbm, dst_hbm,
               nd_hbm, ehat_hbm,
               si_v, di_v, di2_v, av_v, bh_v, ce_v, ms_v, nd_sp, sem):
    c = lax.axis_index("c")
    s = lax.axis_index("s")
    w = s * 2 + c
    zero = jnp.zeros((16,), jnp.float32)

    for p in range(2):
        h0 = p * NHALF
        plsc.subcore_barrier()

        def _zero_buf(r, _):
            for j in range(8):
                ms_v[r, pl.ds(16 * j, 16)] = zero
            return 0

        lax.fori_loop(0, RLEN, _zero_buf, 0)
        for k in range(RCOPIES):
            pltpu.sync_copy(ms_v.at[pl.ds(0, RLEN)],
                            nd_sp.at[pl.ds(s * RBASE + k * RLEN, RLEN)])
        plsc.subcore_barrier()

        def _chunk(t, _):
            chunk = w + t * NWORK

            @pl.when(chunk < NCHUNK)
            def _():
                base = chunk * CHUNK
                pltpu.sync_copy(src_hbm.at[pl.ds(base, CHUNK)], si_v)
                pltpu.sync_copy(dst_hbm.at[pl.ds(base, CHUNK)], di_v)
                cp_av = pltpu.async_copy(av_hbm.at[si_v], av_v, sem)
                cp_bh = pltpu.async_copy(bh_hbm.at[di_v], bh_v, sem)
                cp_ce = pltpu.async_copy(ce_hbm.at[pl.ds(base, CHUNK)], ce_v, sem)
                cp_av.wait()
                cp_bh.wait()
                cp_ce.wait()

                def _idx(i, _):
                    dv = di_v[pl.ds(16 * i, 16)]
                    rel = dv - h0
                    ok = (rel >= 0) & (rel < NHALF)
                    di2_v[pl.ds(16 * i, 16)] = jnp.where(ok, rel, NHALF)
                    return 0

                lax.fori_loop(0, CHUNK // 16, _idx, 0)

                def _row(r, _):
                    for j in range(4):
                        a = av_v[r, pl.ds(16 * j, 16)]
                        vv = av_v[r, pl.ds(64 + 16 * j, 16)]
                        b = bh_v[r, pl.ds(16 * j, 16)]
                        cc = ce_v[r, pl.ds(16 * j, 16)]
                        ehat = a + b + cc
                        if write_ehat and p == 0:
                            ce_v[r, pl.ds(16 * j, 16)] = ehat
                        sg = 1.0 / (1.0 + jnp.exp(-ehat))
                        ms_v[r, pl.ds(64 + 16 * j, 16)] = sg
                        ms_v[r, pl.ds(16 * j, 16)] = sg * vv
                    return 0

                lax.fori_loop(0, CHUNK, _row, 0)

                if write_ehat and p == 0:
                    pltpu.sync_copy(ce_v, ehat_hbm.at[pl.ds(base, CHUNK)])
                pltpu.sync_copy(ms_v, nd_sp.at[di2_v], add=True)
            return 0

        lax.fori_loop(0, TMAX, _chunk, 0)
        plsc.subcore_barrier()

        for k in range(RCOPIES):
            r0 = s * RBASE + k * RLEN
            pltpu.sync_copy(nd_sp.at[pl.ds(r0, RLEN)], ms_v.at[pl.ds(0, RLEN)])
            pltpu.sync_copy(ms_v.at[pl.ds(0, RLEN)],
                            nd_hbm.at[c, pl.ds(h0 + r0, RLEN)])


def _edge_call(write_ehat):
    out_type = [jax.ShapeDtypeStruct((2, N, 2 * HID), jnp.float32)]
    if write_ehat:
        out_type.append(jax.ShapeDtypeStruct((E, HID), jnp.float32))
    body = functools.partial(_edge_body, write_ehat)
    if not write_ehat:
        def body(av, bh, ce, src, dst, nd, *rest):
            return _edge_body(False, av, bh, ce, src, dst, nd, None, *rest)
    return pl.kernel(
        body,
        mesh=plsc.VectorSubcoreMesh(core_axis_name="c", subcore_axis_name="s"),
        out_type=out_type,
        scratch_types=[
            pltpu.VMEM((CHUNK,), jnp.int32),
            pltpu.VMEM((CHUNK,), jnp.int32),
            pltpu.VMEM((CHUNK,), jnp.int32),
            pltpu.VMEM((CHUNK, 2 * HID), jnp.float32),
            pltpu.VMEM((CHUNK, 2 * HID), jnp.float32),
            pltpu.VMEM((CHUNK, HID), jnp.float32),
            pltpu.VMEM((CHUNK, 2 * HID), jnp.float32),
            pltpu.VMEM_SHARED((NHALF + 8, 2 * HID), jnp.float32),
            pltpu.SemaphoreType.DMA,
        ],
    )


def _edge_stage(Ah, Bh, Vh, Ce, src, dst, write_ehat):
    av = jnp.concatenate([Ah, Vh], axis=1)
    bb = jnp.concatenate([Bh, Bh], axis=1)
    outs = _edge_call(write_ehat)(av, bb, Ce, src, dst)
    nd = outs[0][0] + outs[0][1]
    num = nd[:, :HID]
    den = nd[:, HID:] + 1e-6
    return num, den, (outs[1] if write_ehat else None)


def _ln(x):
    m = jnp.mean(x, axis=-1, keepdims=True)
    v = jnp.var(x, axis=-1, keepdims=True)
    return (x - m) / jnp.sqrt(v + 1e-5)


def kernel(x, pos_enc, edge_attr, edge_index, Wn, Wp, b0, We, be, A, B, Cc, U, V, Wq, Wk, Wv, Wo, W1, b1, W2, b2, P1, pb1, P2, pb2, P3, pb3):
    L = A.shape[0]
    src = edge_index[0]
    dst = edge_index[1]
    h = x @ Wn + pos_enc @ Wp + b0
    e = edge_attr @ We + be
    for l in range(L):
        Ah = h @ A[l]
        Bh = h @ B[l]
        Vh = h @ V[l]
        Ce = e @ Cc[l]
        last = (l + 1 == L)
        num, den, e_hat = _edge_stage(Ah, Bh, Vh, Ce, src, dst,
                                      write_ehat=not last)
        h_local = h @ U[l] + num / den
        if not last:
            e = _ln(e + e_hat)
        q = jnp.transpose((h @ Wq[l]).reshape(N, NH, DH), (1, 0, 2))
        k = jnp.transpose((h @ Wk[l]).reshape(N, NH, DH), (1, 0, 2))
        v = jnp.transpose((h @ Wv[l]).reshape(N, NH, DH), (1, 0, 2))
        o = _attention(q, k, v)
        h_attn = jnp.transpose(o, (1, 0, 2)).reshape(N, HID) @ Wo[l]
        h = _ln(h + h_local) + _ln(h + h_attn)
        h = _ln(h + jax.nn.relu(h @ W1[l] + b1[l]) @ W2[l] + b2[l])
    hg = jnp.sum(h, axis=0, keepdims=True)
    z = jax.nn.relu(hg @ P1 + pb1)
    z = jax.nn.relu(z @ P2 + pb2)
    return z @ P3 + pb3


# speedup vs baseline: 2.2444x; 1.2767x over previous
import functools

import jax
import jax.numpy as jnp
from jax import lax
from jax.experimental import pallas as pl
from jax.experimental.pallas import tpu as pltpu
from jax.experimental.pallas import tpu_sc as plsc

N = 10000
E = 320000
HID = 64
NH = 2
DH = HID // NH


BQ = 400


def _attn_body(q_ref, kt_ref, v_ref, o_ref, *, scale):
    q = q_ref[0].astype(jnp.bfloat16)
    kt = kt_ref[0].astype(jnp.bfloat16)
    v = v_ref[0].astype(jnp.bfloat16)
    s = jnp.dot(q, kt, preferred_element_type=jnp.float32) * scale
    m = jnp.max(s, axis=1, keepdims=True)
    p = jnp.exp(s - m)
    l = jnp.sum(p, axis=1, keepdims=True)
    o = jnp.dot(p.astype(jnp.bfloat16), v, preferred_element_type=jnp.float32) / l
    o_ref[0] = o


def _attention(q, k, v):
    kt = jnp.swapaxes(k, 1, 2)
    return pl.pallas_call(
        functools.partial(_attn_body, scale=float(1.0 / (DH ** 0.5))),
        grid=(NH, N // BQ),
        in_specs=[
            pl.BlockSpec((1, BQ, DH), lambda h, i: (h, i, 0)),
            pl.BlockSpec((1, DH, N), lambda h, i: (h, 0, 0)),
            pl.BlockSpec((1, N, DH), lambda h, i: (h, 0, 0)),
        ],
        out_specs=pl.BlockSpec((1, BQ, DH), lambda h, i: (h, i, 0)),
        out_shape=jax.ShapeDtypeStruct((NH, N, DH), jnp.float32),
    )(q, kt, v)


CHUNK = 128
NCHUNK = E // CHUNK
NWORK = 32
TMAX = -(-NCHUNK // NWORK)
RBASE = 624
RLEN = 128
RCOPIES = 5


def _edge_body(write_ehat, av_hbm, bh_hbm, ce_hbm, src_hbm, dst_hbm,
               nd_hbm, ehat_hbm,
               si_v, di_v, av_v, bh_v, ce_v, nd_sp, sem):
    c = lax.axis_index("c")
    s = lax.axis_index("s")
    w = s * 2 + c
    zero = jnp.zeros((16,), jnp.float32)

    def _zero_buf(r, _):
        for j in range(8):
            av_v[r, pl.ds(16 * j, 16)] = zero
        return 0

    lax.fori_loop(0, RLEN, _zero_buf, 0)
    for k in range(RCOPIES):
        pltpu.sync_copy(av_v.at[pl.ds(0, RLEN)],
                        nd_sp.at[pl.ds(s * RBASE + k * RLEN, RLEN)])
    plsc.subcore_barrier()

    def _chunk(t, _):
        chunk = w + t * NWORK

        @pl.when(chunk < NCHUNK)
        def _():
            base = chunk * CHUNK
            pltpu.sync_copy(src_hbm.at[pl.ds(base, CHUNK)], si_v)
            pltpu.sync_copy(dst_hbm.at[pl.ds(base, CHUNK)], di_v)
            cp_av = pltpu.async_copy(av_hbm.at[si_v], av_v, sem)
            cp_bh = pltpu.async_copy(bh_hbm.at[di_v], bh_v, sem)
            cp_ce = pltpu.async_copy(ce_hbm.at[pl.ds(base, CHUNK)], ce_v, sem)
            cp_av.wait()
            cp_bh.wait()
            cp_ce.wait()

            def _row(r, _):
                for j in range(4):
                    a = av_v[r, pl.ds(16 * j, 16)]
                    vv = av_v[r, pl.ds(64 + 16 * j, 16)]
                    b = bh_v[r, pl.ds(16 * j, 16)]
                    cc = ce_v[r, pl.ds(16 * j, 16)]
                    ehat = a + b + cc
                    if write_ehat:
                        ce_v[r, pl.ds(16 * j, 16)] = ehat
                    sg = 1.0 / (1.0 + jnp.exp(-ehat))
                    av_v[r, pl.ds(64 + 16 * j, 16)] = sg
                    av_v[r, pl.ds(16 * j, 16)] = sg * vv
                return 0

            lax.fori_loop(0, CHUNK, _row, 0)

            if write_ehat:
                pltpu.sync_copy(ce_v, ehat_hbm.at[pl.ds(base, CHUNK)])
            pltpu.sync_copy(av_v, nd_sp.at[di_v], add=True)
        return 0

    lax.fori_loop(0, TMAX, _chunk, 0)
    plsc.subcore_barrier()

    for k in range(RCOPIES):
        r0 = s * RBASE + k * RLEN
        pltpu.sync_copy(nd_sp.at[pl.ds(r0, RLEN)], av_v.at[pl.ds(0, RLEN)])
        pltpu.sync_copy(av_v.at[pl.ds(0, RLEN)],
                        nd_hbm.at[c, pl.ds(r0, RLEN)])


def _edge_call(write_ehat):
    out_type = [jax.ShapeDtypeStruct((2, N, 2 * HID), jnp.float32)]
    if write_ehat:
        out_type.append(jax.ShapeDtypeStruct((E, HID), jnp.float32))
    body = functools.partial(_edge_body, write_ehat)
    if not write_ehat:
        def body(av, bh, ce, src, dst, nd, *rest):
            return _edge_body(False, av, bh, ce, src, dst, nd, None, *rest)
    return pl.kernel(
        body,
        mesh=plsc.VectorSubcoreMesh(core_axis_name="c", subcore_axis_name="s"),
        out_type=out_type,
        scratch_types=[
            pltpu.VMEM((CHUNK,), jnp.int32),
            pltpu.VMEM((CHUNK,), jnp.int32),
            pltpu.VMEM((CHUNK, 2 * HID), jnp.float32),
            pltpu.VMEM((CHUNK, 2 * HID), jnp.float32),
            pltpu.VMEM((CHUNK, HID), jnp.float32),
            pltpu.VMEM_SHARED((N + 8, 2 * HID), jnp.float32),
            pltpu.SemaphoreType.DMA,
        ],
    )


def _edge_stage(Ah, Bh, Vh, Ce, src, dst, write_ehat):
    av = jnp.concatenate([Ah, Vh], axis=1)
    bb = jnp.concatenate([Bh, Bh], axis=1)
    outs = _edge_call(write_ehat)(av, bb, Ce, src, dst)
    nd = outs[0][0] + outs[0][1]
    num = nd[:, :HID]
    den = nd[:, HID:] + 1e-6
    return num, den, (outs[1] if write_ehat else None)


def _ln(x):
    m = jnp.mean(x, axis=-1, keepdims=True)
    v = jnp.var(x, axis=-1, keepdims=True)
    return (x - m) / jnp.sqrt(v + 1e-5)


def kernel(x, pos_enc, edge_attr, edge_index, Wn, Wp, b0, We, be, A, B, Cc, U, V, Wq, Wk, Wv, Wo, W1, b1, W2, b2, P1, pb1, P2, pb2, P3, pb3):
    L = A.shape[0]
    src = edge_index[0]
    dst = edge_index[1]
    h = x @ Wn + pos_enc @ Wp + b0
    e = edge_attr @ We + be
    for l in range(L):
        Ah = h @ A[l]
        Bh = h @ B[l]
        Vh = h @ V[l]
        Ce = e @ Cc[l]
        last = (l + 1 == L)
        num, den, e_hat = _edge_stage(Ah, Bh, Vh, Ce, src, dst,
                                      write_ehat=not last)
        h_local = h @ U[l] + num / den
        if not last:
            e = _ln(e + e_hat)
        q = jnp.transpose((h @ Wq[l]).reshape(N, NH, DH), (1, 0, 2))
        k = jnp.transpose((h @ Wk[l]).reshape(N, NH, DH), (1, 0, 2))
        v = jnp.transpose((h @ Wv[l]).reshape(N, NH, DH), (1, 0, 2))
        o = _attention(q, k, v)
        h_attn = jnp.transpose(o, (1, 0, 2)).reshape(N, HID) @ Wo[l]
        h = _ln(h + h_local) + _ln(h + h_attn)
        h = _ln(h + jax.nn.relu(h @ W1[l] + b1[l]) @ W2[l] + b2[l])
    hg = jnp.sum(h, axis=0, keepdims=True)
    z = jax.nn.relu(hg @ P1 + pb1)
    z = jax.nn.relu(z @ P2 + pb2)
    return z @ P3 + pb3


# speedup vs baseline: 2.2453x; 1.0004x over previous
import functools

import jax
import jax.numpy as jnp
from jax import lax
from jax.experimental import pallas as pl
from jax.experimental.pallas import tpu as pltpu
from jax.experimental.pallas import tpu_sc as plsc

N = 10000
E = 320000
HID = 64
NH = 2
DH = HID // NH


BQ = 400


def _attn_body(q_ref, kt_ref, v_ref, o_ref, *, scale):
    q = q_ref[0].astype(jnp.bfloat16)
    kt = kt_ref[0].astype(jnp.bfloat16)
    v = v_ref[0].astype(jnp.bfloat16)
    s = jnp.dot(q, kt, preferred_element_type=jnp.float32) * scale
    m = jnp.max(s, axis=1, keepdims=True)
    p = jnp.exp(s - m)
    l = jnp.sum(p, axis=1, keepdims=True)
    o = jnp.dot(p.astype(jnp.bfloat16), v, preferred_element_type=jnp.float32) / l
    o_ref[0] = o


def _attention(q, k, v):
    kt = jnp.swapaxes(k, 1, 2)
    return pl.pallas_call(
        functools.partial(_attn_body, scale=float(1.0 / (DH ** 0.5))),
        grid=(NH, N // BQ),
        in_specs=[
            pl.BlockSpec((1, BQ, DH), lambda h, i: (h, i, 0)),
            pl.BlockSpec((1, DH, N), lambda h, i: (h, 0, 0)),
            pl.BlockSpec((1, N, DH), lambda h, i: (h, 0, 0)),
        ],
        out_specs=pl.BlockSpec((1, BQ, DH), lambda h, i: (h, i, 0)),
        out_shape=jax.ShapeDtypeStruct((NH, N, DH), jnp.float32),
    )(q, kt, v)


CHUNK = 128
NCHUNK = E // CHUNK
NWORK = 32
TMAX = -(-NCHUNK // NWORK)
RBASE = 624
RLEN = 128
RCOPIES = 5


def _edge_body(write_ehat, av_hbm, bh_hbm, ce_hbm, src_hbm, dst_hbm,
               nd_hbm, ehat_hbm,
               si_v, di_v, av_v, bh_v, ce_v, nd_sp, sem):
    c = lax.axis_index("c")
    s = lax.axis_index("s")
    w = s * 2 + c
    zero = jnp.zeros((16,), jnp.float32)

    def _zero_buf(r, _):
        for j in range(8):
            av_v[r, pl.ds(16 * j, 16)] = zero
        return 0

    lax.fori_loop(0, RLEN, _zero_buf, 0)
    for k in range(RCOPIES):
        pltpu.sync_copy(av_v.at[pl.ds(0, RLEN)],
                        nd_sp.at[pl.ds(s * RBASE + k * RLEN, RLEN)])
    plsc.subcore_barrier()

    def _chunk(t, _):
        chunk = w + t * NWORK

        @pl.when(chunk < NCHUNK)
        def _():
            base = chunk * CHUNK
            pltpu.sync_copy(src_hbm.at[pl.ds(base, CHUNK)], si_v)
            pltpu.sync_copy(dst_hbm.at[pl.ds(base, CHUNK)], di_v)
            cp_av = pltpu.async_copy(av_hbm.at[si_v], av_v, sem)
            cp_bh = pltpu.async_copy(bh_hbm.at[di_v], bh_v, sem)
            cp_ce = pltpu.async_copy(ce_hbm.at[pl.ds(base, CHUNK)], ce_v, sem)
            cp_av.wait()
            cp_bh.wait()
            cp_ce.wait()

            def _row(r, _):
                for j in range(4):
                    a = av_v[r, pl.ds(16 * j, 16)]
                    vv = av_v[r, pl.ds(64 + 16 * j, 16)]
                    b = bh_v[r, pl.ds(16 * j, 16)]
                    cc = ce_v[r, pl.ds(16 * j, 16)]
                    ehat = a + b + cc
                    if write_ehat:
                        ce_v[r, pl.ds(16 * j, 16)] = ehat
                    sg = 1.0 / (1.0 + jnp.exp(-ehat))
                    av_v[r, pl.ds(64 + 16 * j, 16)] = sg
                    av_v[r, pl.ds(16 * j, 16)] = sg * vv
                return 0

            lax.fori_loop(0, CHUNK, _row, 0)

            if write_ehat:
                pltpu.sync_copy(ce_v, ehat_hbm.at[pl.ds(base, CHUNK)])
            pltpu.sync_copy(av_v, nd_sp.at[di_v], add=True)
        return 0

    lax.fori_loop(0, TMAX, _chunk, 0)
    plsc.subcore_barrier()

    for k in range(RCOPIES):
        r0 = s * RBASE + k * RLEN
        pltpu.sync_copy(nd_sp.at[pl.ds(r0, RLEN)], av_v.at[pl.ds(0, RLEN)])
        pltpu.sync_copy(av_v.at[pl.ds(0, RLEN)],
                        nd_hbm.at[c, pl.ds(r0, RLEN)])


def _edge_call(write_ehat):
    out_type = [jax.ShapeDtypeStruct((2, N, 2 * HID), jnp.float32)]
    if write_ehat:
        out_type.append(jax.ShapeDtypeStruct((E, HID), jnp.float32))
    body = functools.partial(_edge_body, write_ehat)
    if not write_ehat:
        def body(av, bh, ce, src, dst, nd, *rest):
            return _edge_body(False, av, bh, ce, src, dst, nd, None, *rest)
    return pl.kernel(
        body,
        mesh=plsc.VectorSubcoreMesh(core_axis_name="c", subcore_axis_name="s"),
        out_type=out_type,
        scratch_types=[
            pltpu.VMEM((CHUNK,), jnp.int32),
            pltpu.VMEM((CHUNK,), jnp.int32),
            pltpu.VMEM((CHUNK, 2 * HID), jnp.float32),
            pltpu.VMEM((CHUNK, 2 * HID), jnp.float32),
            pltpu.VMEM((CHUNK, HID), jnp.float32),
            pltpu.VMEM_SHARED((N + 8, 2 * HID), jnp.float32),
            pltpu.SemaphoreType.DMA,
        ],
    )


def _edge_stage(Ah, Bh, Vh, Ce, src, dst, write_ehat):
    av = jnp.concatenate([Ah, Vh], axis=1)
    bb = jnp.concatenate([Bh, Bh], axis=1)
    outs = _edge_call(write_ehat)(av, bb, Ce, src, dst)
    nd = outs[0][0] + outs[0][1]
    num = nd[:, :HID]
    den = nd[:, HID:] + 1e-6
    return num, den, (outs[1] if write_ehat else None)


def _ln(x):
    m = jnp.mean(x, axis=-1, keepdims=True)
    v = jnp.var(x, axis=-1, keepdims=True)
    return (x - m) / jnp.sqrt(v + 1e-5)


def kernel(x, pos_enc, edge_attr, edge_index, Wn, Wp, b0, We, be, A, B, Cc, U, V, Wq, Wk, Wv, Wo, W1, b1, W2, b2, P1, pb1, P2, pb2, P3, pb3):
    L = A.shape[0]
    src = edge_index[0]
    dst = edge_index[1]
    h = x @ Wn + pos_enc @ Wp + b0
    e = edge_attr @ We + be
    for l in range(L):
        Ah = h @ A[l]
        Bh = h @ B[l]
        Vh = h @ V[l]
        Ce = e @ Cc[l]
        last = (l + 1 == L)
        num, den, e_hat = _edge_stage(Ah, Bh, Vh, Ce, src, dst,
                                      write_ehat=not last)
        q = jnp.transpose((h @ Wq[l]).reshape(N, NH, DH), (1, 0, 2))
        k = jnp.transpose((h @ Wk[l]).reshape(N, NH, DH), (1, 0, 2))
        v = jnp.transpose((h @ Wv[l]).reshape(N, NH, DH), (1, 0, 2))
        o = _attention(q, k, v)
        h_attn = jnp.transpose(o, (1, 0, 2)).reshape(N, HID) @ Wo[l]
        h_local = h @ U[l] + num / den
        if not last:
            e = _ln(e + e_hat)
        h = _ln(h + h_local) + _ln(h + h_attn)
        h = _ln(h + jax.nn.relu(h @ W1[l] + b1[l]) @ W2[l] + b2[l])
    hg = jnp.sum(h, axis=0, keepdims=True)
    z = jax.nn.relu(hg @ P1 + pb1)
    z = jax.nn.relu(z @ P2 + pb2)
    return z @ P3 + pb3


# speedup vs baseline: 2.3075x; 1.0277x over previous
import functools

import jax
import jax.numpy as jnp
from jax import lax
from jax.experimental import pallas as pl
from jax.experimental.pallas import tpu as pltpu
from jax.experimental.pallas import tpu_sc as plsc

N = 10000
E = 320000
HID = 64
NH = 2
DH = HID // NH


BQ = 400


def _attn_body(q_ref, kt_ref, v_ref, o_ref):
    q = q_ref[0].astype(jnp.bfloat16)
    kt = kt_ref[0].astype(jnp.bfloat16)
    v = v_ref[0].astype(jnp.bfloat16)
    s = jnp.dot(q, kt, preferred_element_type=jnp.float32)
    p = jnp.exp2(s)
    l = jnp.sum(p, axis=1, keepdims=True)
    o = jnp.dot(p.astype(jnp.bfloat16), v, preferred_element_type=jnp.float32) / l
    o_ref[0] = o


def _attention(q, k, v):
    kt = jnp.swapaxes(k, 1, 2)
    q = q * jnp.float32(1.4426950408889634 / (DH ** 0.5))
    return pl.pallas_call(
        _attn_body,
        grid=(NH, N // BQ),
        in_specs=[
            pl.BlockSpec((1, BQ, DH), lambda h, i: (h, i, 0)),
            pl.BlockSpec((1, DH, N), lambda h, i: (h, 0, 0)),
            pl.BlockSpec((1, N, DH), lambda h, i: (h, 0, 0)),
        ],
        out_specs=pl.BlockSpec((1, BQ, DH), lambda h, i: (h, i, 0)),
        out_shape=jax.ShapeDtypeStruct((NH, N, DH), jnp.float32),
    )(q, kt, v)


CHUNK = 128
NCHUNK = E // CHUNK
NWORK = 32
TMAX = -(-NCHUNK // NWORK)
RBASE = 624
RLEN = 128
RCOPIES = 5


def _edge_body(write_ehat, av_hbm, bh_hbm, ce_hbm, src_hbm, dst_hbm,
               nd_hbm, ehat_hbm,
               si_v, di_v, av_v, bh_v, ce_v, nd_sp, sem):
    c = lax.axis_index("c")
    s = lax.axis_index("s")
    w = s * 2 + c
    zero = jnp.zeros((16,), jnp.float32)

    def _zero_buf(r, _):
        for j in range(8):
            av_v[r, pl.ds(16 * j, 16)] = zero
        return 0

    lax.fori_loop(0, RLEN, _zero_buf, 0)
    for k in range(RCOPIES):
        pltpu.sync_copy(av_v.at[pl.ds(0, RLEN)],
                        nd_sp.at[pl.ds(s * RBASE + k * RLEN, RLEN)])
    plsc.subcore_barrier()

    def _chunk(t, _):
        chunk = w + t * NWORK

        @pl.when(chunk < NCHUNK)
        def _():
            base = chunk * CHUNK
            pltpu.sync_copy(src_hbm.at[pl.ds(base, CHUNK)], si_v)
            pltpu.sync_copy(dst_hbm.at[pl.ds(base, CHUNK)], di_v)
            cp_av = pltpu.async_copy(av_hbm.at[si_v], av_v, sem)
            cp_bh = pltpu.async_copy(bh_hbm.at[di_v], bh_v, sem)
            cp_ce = pltpu.async_copy(ce_hbm.at[pl.ds(base, CHUNK)], ce_v, sem)
            cp_av.wait()
            cp_bh.wait()
            cp_ce.wait()

            def _row(r, _):
                for j in range(4):
                    a = av_v[r, pl.ds(16 * j, 16)]
                    vv = av_v[r, pl.ds(64 + 16 * j, 16)]
                    b = bh_v[r, pl.ds(16 * j, 16)]
                    cc = ce_v[r, pl.ds(16 * j, 16)]
                    ehat = a + b + cc
                    if write_ehat:
                        ce_v[r, pl.ds(16 * j, 16)] = ehat
                    sg = 1.0 / (1.0 + jnp.exp(-ehat))
                    av_v[r, pl.ds(64 + 16 * j, 16)] = sg
                    av_v[r, pl.ds(16 * j, 16)] = sg * vv
                return 0

            lax.fori_loop(0, CHUNK, _row, 0)

            if write_ehat:
                pltpu.sync_copy(ce_v, ehat_hbm.at[pl.ds(base, CHUNK)])
            pltpu.sync_copy(av_v, nd_sp.at[di_v], add=True)
        return 0

    lax.fori_loop(0, TMAX, _chunk, 0)
    plsc.subcore_barrier()

    for k in range(RCOPIES):
        r0 = s * RBASE + k * RLEN
        pltpu.sync_copy(nd_sp.at[pl.ds(r0, RLEN)], av_v.at[pl.ds(0, RLEN)])
        pltpu.sync_copy(av_v.at[pl.ds(0, RLEN)],
                        nd_hbm.at[c, pl.ds(r0, RLEN)])


def _edge_call(write_ehat):
    out_type = [jax.ShapeDtypeStruct((2, N, 2 * HID), jnp.float32)]
    if write_ehat:
        out_type.append(jax.ShapeDtypeStruct((E, HID), jnp.float32))
    body = functools.partial(_edge_body, write_ehat)
    if not write_ehat:
        def body(av, bh, ce, src, dst, nd, *rest):
            return _edge_body(False, av, bh, ce, src, dst, nd, None, *rest)
    return pl.kernel(
        body,
        mesh=plsc.VectorSubcoreMesh(core_axis_name="c", subcore_axis_name="s"),
        out_type=out_type,
        scratch_types=[
            pltpu.VMEM((CHUNK,), jnp.int32),
            pltpu.VMEM((CHUNK,), jnp.int32),
            pltpu.VMEM((CHUNK, 2 * HID), jnp.float32),
            pltpu.VMEM((CHUNK, 2 * HID), jnp.float32),
            pltpu.VMEM((CHUNK, HID), jnp.float32),
            pltpu.VMEM_SHARED((N + 8, 2 * HID), jnp.float32),
            pltpu.SemaphoreType.DMA,
        ],
    )


def _edge_stage(Ah, Bh, Vh, Ce, src, dst, write_ehat):
    av = jnp.concatenate([Ah, Vh], axis=1)
    bb = jnp.concatenate([Bh, Bh], axis=1)
    outs = _edge_call(write_ehat)(av, bb, Ce, src, dst)
    nd = outs[0][0] + outs[0][1]
    num = nd[:, :HID]
    den = nd[:, HID:] + 1e-6
    return num, den, (outs[1] if write_ehat else None)


def _ln(x):
    m = jnp.mean(x, axis=-1, keepdims=True)
    v = jnp.var(x, axis=-1, keepdims=True)
    return (x - m) / jnp.sqrt(v + 1e-5)


def kernel(x, pos_enc, edge_attr, edge_index, Wn, Wp, b0, We, be, A, B, Cc, U, V, Wq, Wk, Wv, Wo, W1, b1, W2, b2, P1, pb1, P2, pb2, P3, pb3):
    L = A.shape[0]
    src = edge_index[0]
    dst = edge_index[1]
    h = x @ Wn + pos_enc @ Wp + b0
    e = edge_attr @ We + be
    for l in range(L):
        Ah = h @ A[l]
        Bh = h @ B[l]
        Vh = h @ V[l]
        Ce = e @ Cc[l]
        last = (l + 1 == L)
        num, den, e_hat = _edge_stage(Ah, Bh, Vh, Ce, src, dst,
                                      write_ehat=not last)
        q = jnp.transpose((h @ Wq[l]).reshape(N, NH, DH), (1, 0, 2))
        k = jnp.transpose((h @ Wk[l]).reshape(N, NH, DH), (1, 0, 2))
        v = jnp.transpose((h @ Wv[l]).reshape(N, NH, DH), (1, 0, 2))
        o = _attention(q, k, v)
        h_attn = jnp.transpose(o, (1, 0, 2)).reshape(N, HID) @ Wo[l]
        h_local = h @ U[l] + num / den
        if not last:
            e = _ln(e + e_hat)
        h = _ln(h + h_local) + _ln(h + h_attn)
        h = _ln(h + jax.nn.relu(h @ W1[l] + b1[l]) @ W2[l] + b2[l])
    hg = jnp.sum(h, axis=0, keepdims=True)
    z = jax.nn.relu(hg @ P1 + pb1)
    z = jax.nn.relu(z @ P2 + pb2)
    return z @ P3 + pb3
